# baseline (speedup 1.0000x reference)
import functools

import jax
import jax.numpy as jnp
from jax import lax
from jax.experimental import pallas as pl
from jax.experimental.pallas import tpu as pltpu
from jax.experimental.pallas import tpu_sc as plsc

_NC = 2
_NS = 16
_NW = _NC * _NS
_L = 16

_B = 4
_T = 4096
_F = 1024
_GS = 4
_TN = _T // _GS
_WINDOWS = _B * _TN
_RPW_M = _WINDOWS // _NW

_S = 1024


def _sc_mask_body(m_hbm, mout_hbm, m_v, mo_v):
    wid = lax.axis_index("s") * _NC + lax.axis_index("c")
    base = wid * _RPW_M

    for k in range(_GS):
        pltpu.sync_copy(m_hbm.at[pl.ds(k * _WINDOWS + base, _RPW_M)], m_v.at[k])

    def mask_blk(j, _):
        acc = m_v[0, pl.ds(j * _L, _L)]
        for k in range(1, _GS):
            acc = jnp.minimum(acc, m_v[k, pl.ds(j * _L, _L)])
        mo_v[pl.ds(j * _L, _L)] = acc
        return 0

    lax.fori_loop(0, _RPW_M // _L, mask_blk, 0, unroll=True)
    pltpu.sync_copy(mo_v, mout_hbm.at[pl.ds(base, _RPW_M)])


_sc_mask = functools.partial(
    pl.kernel,
    out_type=jax.ShapeDtypeStruct((_WINDOWS,), jnp.int32),
    mesh=plsc.VectorSubcoreMesh(core_axis_name="c", subcore_axis_name="s"),
    scratch_types=[
        pltpu.VMEM((_GS, _RPW_M), jnp.int32),
        pltpu.VMEM((_RPW_M,), jnp.int32),
    ],
)(_sc_mask_body)


def _tc_body(x_ref, o_ref):
    x = x_ref[0].reshape(_S // 2, 2, _F)
    r1 = x[:, 0, :] + x[:, 1, :]
    r2 = r1.reshape(_S // 4, 2, _F)
    o_ref[0] = (r2[:, 0, :] + r2[:, 1, :]) * (1.0 / _GS)


def kernel(dense_x, dense_padding_mask):
    bsz, tsz, fsz = dense_x.shape

    m4 = (
        dense_padding_mask.reshape(_WINDOWS, _GS)
        .astype(jnp.int32)
        .T.reshape(_GS * _WINDOWS)
    )
    mout = _sc_mask(m4)

    out = pl.pallas_call(
        _tc_body,
        grid=(_B, _T // _S),
        in_specs=[pl.BlockSpec((1, _S, _F), lambda b, j: (b, j, 0))],
        out_specs=pl.BlockSpec((1, _S // _GS, _F), lambda b, j: (b, j, 0)),
        out_shape=jax.ShapeDtypeStruct((_B, _TN, _F), jnp.float32),
    )(dense_x)

    return (out, mout.reshape(bsz, _TN).astype(jnp.bool_))


# speedup vs baseline: 1.0560x; 1.0560x over previous
import functools

import jax
import jax.numpy as jnp
from jax import lax
from jax.experimental import pallas as pl
from jax.experimental.pallas import tpu as pltpu
from jax.experimental.pallas import tpu_sc as plsc

_NC = 2
_NS = 16
_NW = _NC * _NS
_L = 16

_B = 4
_T = 4096
_F = 1024
_GS = 4
_TN = _T // _GS
_WINDOWS = _B * _TN
_RPW_M = _WINDOWS // _NW

_S = 4096


def _sc_mask_body(m_hbm, mout_hbm, m_v, mo_v):
    wid = lax.axis_index("s") * _NC + lax.axis_index("c")
    base = wid * _RPW_M

    for k in range(_GS):
        pltpu.sync_copy(m_hbm.at[pl.ds(k * _WINDOWS + base, _RPW_M)], m_v.at[k])

    def mask_blk(j, _):
        acc = m_v[0, pl.ds(j * _L, _L)]
        for k in range(1, _GS):
            acc = jnp.minimum(acc, m_v[k, pl.ds(j * _L, _L)])
        mo_v[pl.ds(j * _L, _L)] = acc
        return 0

    lax.fori_loop(0, _RPW_M // _L, mask_blk, 0, unroll=True)
    pltpu.sync_copy(mo_v, mout_hbm.at[pl.ds(base, _RPW_M)])


_sc_mask = functools.partial(
    pl.kernel,
    out_type=jax.ShapeDtypeStruct((_WINDOWS,), jnp.int32),
    mesh=plsc.VectorSubcoreMesh(core_axis_name="c", subcore_axis_name="s"),
    scratch_types=[
        pltpu.VMEM((_GS, _RPW_M), jnp.int32),
        pltpu.VMEM((_RPW_M,), jnp.int32),
    ],
)(_sc_mask_body)


def _tc_body(x_ref, o_ref):
    x = x_ref[0].reshape(_S // 2, 2, _F)
    r1 = x[:, 0, :] + x[:, 1, :]
    r2 = r1.reshape(_S // 4, 2, _F)
    o_ref[0] = (r2[:, 0, :] + r2[:, 1, :]) * (1.0 / _GS)


def kernel(dense_x, dense_padding_mask):
    bsz, tsz, fsz = dense_x.shape

    m4 = (
        dense_padding_mask.reshape(_WINDOWS, _GS)
        .astype(jnp.int32)
        .T.reshape(_GS * _WINDOWS)
    )
    mout = _sc_mask(m4)

    out = pl.pallas_call(
        _tc_body,
        grid=(_B, _T // _S),
        in_specs=[pl.BlockSpec((1, _S, _F), lambda b, j: (b, j, 0))],
        out_specs=pl.BlockSpec((1, _S // _GS, _F), lambda b, j: (b, j, 0)),
        out_shape=jax.ShapeDtypeStruct((_B, _TN, _F), jnp.float32),
    )(dense_x)

    return (out, mout.reshape(bsz, _TN).astype(jnp.bool_))
